# Initial kernel scaffold; baseline (speedup 1.0000x reference)
import jax
import jax.numpy as jnp
from jax.experimental import pallas as pl


def kernel(input_tokens, table):
    raise NotImplementedError("write your pallas kernel here")


# speedup vs baseline: 6.1282x; 6.1282x over previous
import functools

import jax
import jax.numpy as jnp
from jax import lax
from jax.experimental import pallas as pl
from jax.experimental.pallas import tpu as pltpu
from jax.experimental.pallas import tpu_sc as plsc

_INFO = plsc.get_sparse_core_info()
_NC = _INFO.num_cores
_NS = _INFO.num_subcores
_NW = _NC * _NS
_G = 128
_K = 8
_CH = _K * _G


def _make_gather(n_rows, vocab, dim):
    assert n_rows % (_NW * _CH) == 0
    per_w = n_rows // _NW
    n_chunks = per_w // _CH
    mesh = plsc.VectorSubcoreMesh(core_axis_name="c", subcore_axis_name="s")

    @functools.partial(
        pl.kernel,
        out_type=jax.ShapeDtypeStruct((n_rows, dim), jnp.float32),
        mesh=mesh,
        scratch_types=[
            pltpu.VMEM((_K, _G), jnp.int32),
            pltpu.VMEM((_CH, dim), jnp.float32),
            pltpu.SemaphoreType.DMA,
        ],
        compiler_params=pltpu.CompilerParams(use_tc_tiling_on_sc=False),
    )
    def gather_kernel(idx_hbm, table_hbm, out_hbm, idx_v, rows_v, gsem):
        c = lax.axis_index("c")
        s = lax.axis_index("s")
        wid = s * _NC + c
        base_w = wid * per_w

        def chunk_body(g, carry):
            pltpu.sync_copy(idx_hbm.at[wid, g], idx_v)
            copies = [
                pltpu.async_copy(
                    table_hbm.at[idx_v.at[j]],
                    rows_v.at[pl.ds(j * _G, _G)],
                    gsem,
                )
                for j in range(_K)
            ]
            for cp in copies:
                cp.wait()
            pltpu.sync_copy(rows_v, out_hbm.at[pl.ds(base_w + g * _CH, _CH)])
            return carry

        lax.fori_loop(0, n_chunks, chunk_body, 0)

    return gather_kernel


def kernel(input_tokens, table):
    batch, hist = input_tokens.shape
    vocab, dim = table.shape
    n_rows = batch * hist
    idx = input_tokens.reshape(_NW, n_rows // (_NW * _CH), _K, _G)
    idx = idx.astype(jnp.int32)
    out = _make_gather(n_rows, vocab, dim)(idx, table)
    return out.reshape(batch, hist, dim)


# speedup vs baseline: 6.4799x; 1.0574x over previous
import functools

import jax
import jax.numpy as jnp
from jax import lax
from jax.experimental import pallas as pl
from jax.experimental.pallas import tpu as pltpu
from jax.experimental.pallas import tpu_sc as plsc

_INFO = plsc.get_sparse_core_info()
_NC = _INFO.num_cores
_NS = _INFO.num_subcores
_NW = _NC * _NS
_G = 128
_K = 8
_CH = _K * _G


def _make_gather(n_rows, vocab, dim):
    assert n_rows % (_NW * _CH) == 0
    per_w = n_rows // _NW
    n_chunks = per_w // _CH
    mesh = plsc.VectorSubcoreMesh(core_axis_name="c", subcore_axis_name="s")

    @functools.partial(
        pl.kernel,
        out_type=jax.ShapeDtypeStruct((n_rows, dim), jnp.float32),
        mesh=mesh,
        scratch_types=[
            pltpu.VMEM((2, _K, _G), jnp.int32),
            pltpu.VMEM((2, _CH, dim), jnp.float32),
            pltpu.SemaphoreType.DMA,
            pltpu.SemaphoreType.DMA,
            pltpu.SemaphoreType.DMA,
        ],
        compiler_params=pltpu.CompilerParams(use_tc_tiling_on_sc=False),
    )
    def gather_kernel(idx_hbm, table_hbm, out_hbm, idx_v, rows_v, isem, gsem, osem):
        c = lax.axis_index("c")
        s = lax.axis_index("s")
        wid = s * _NC + c
        base_w = wid * per_w

        def wait_idx(buf):
            pltpu.make_async_copy(idx_hbm.at[0, 0], idx_v.at[buf], isem).wait()

        def wait_rows(buf):
            pltpu.make_async_copy(
                table_hbm.at[pl.ds(0, _CH)], rows_v.at[buf], gsem
            ).wait()

        def wait_out(buf):
            pltpu.make_async_copy(
                rows_v.at[buf], out_hbm.at[pl.ds(0, _CH)], osem
            ).wait()

        def fire_gathers(buf):
            for j in range(_K):
                pltpu.async_copy(
                    table_hbm.at[idx_v.at[buf, j]],
                    rows_v.at[buf, pl.ds(j * _G, _G)],
                    gsem,
                )

        pltpu.async_copy(idx_hbm.at[wid, 0], idx_v.at[0], isem)
        wait_idx(0)
        fire_gathers(0)
        pltpu.async_copy(idx_hbm.at[wid, 1], idx_v.at[1], isem)

        def chunk_body(g, carry):
            cur = lax.rem(g, 2)
            prv = 1 - cur
            wait_rows(prv)
            pltpu.async_copy(
                rows_v.at[prv],
                out_hbm.at[pl.ds(base_w + (g - 1) * _CH, _CH)],
                osem,
            )
            @pl.when(g + 1 < n_chunks)
            def _():
                pltpu.async_copy(idx_hbm.at[wid, g + 1], idx_v.at[prv], isem)

            wait_idx(cur)
            @pl.when(g >= 2)
            def _():
                wait_out(cur)
            fire_gathers(cur)
            return carry

        lax.fori_loop(1, n_chunks, chunk_body, 0)

        last = n_chunks - 1
        lbuf = last % 2
        wait_rows(lbuf)
        pltpu.async_copy(
            rows_v.at[lbuf], out_hbm.at[pl.ds(base_w + last * _CH, _CH)], osem
        )
        wait_out(0)
        wait_out(1)

    return gather_kernel


def kernel(input_tokens, table):
    batch, hist = input_tokens.shape
    vocab, dim = table.shape
    n_rows = batch * hist
    idx = input_tokens.reshape(_NW, n_rows // (_NW * _CH), _K, _G)
    idx = idx.astype(jnp.int32)
    out = _make_gather(n_rows, vocab, dim)(idx, table)
    return out.reshape(batch, hist, dim)
